# Initial kernel scaffold; baseline (speedup 1.0000x reference)
import jax
import jax.numpy as jnp
from jax.experimental import pallas as pl


def kernel(input, weight):
    raise NotImplementedError("write your pallas kernel here")


# speedup vs baseline: 15.4118x; 15.4118x over previous
import functools

import jax
import jax.numpy as jnp
from jax import lax
from jax.experimental import pallas as pl
from jax.experimental.pallas import tpu as pltpu
from jax.experimental.pallas import tpu_sc as plsc

NC = 2
NS = 16
NW = NC * NS
GROUP = 128


@functools.partial(jax.jit, static_argnames=("n_batches",))
def _emb_lookup(idx2d, table, n_batches):
    n_groups_total, _ = idx2d.shape
    _, V, D = table.shape
    n_rows = n_groups_total * GROUP
    G = n_groups_total // NW
    per_tile = G * GROUP
    K = 8
    n_super = G // K
    tiles_per_batch = NW // n_batches

    mesh = plsc.VectorSubcoreMesh(core_axis_name="c", subcore_axis_name="s")

    def body(idx_hbm, tab_hbm, out_hbm, idx_v, rows_v, gsem):
        c = lax.axis_index("c")
        s = lax.axis_index("s")
        wid = s * NC + c
        b = wid // tiles_per_batch
        gbase = wid * G
        rbase = wid * per_tile

        pltpu.sync_copy(idx_hbm.at[pl.ds(gbase, G)], idx_v)

        def super_body(p, _):
            copies = []
            for k in range(K):
                copies.append(
                    pltpu.async_copy(
                        tab_hbm.at[b].at[idx_v.at[p * K + k]],
                        rows_v.at[pl.ds(k * GROUP, GROUP)],
                        gsem,
                    )
                )
            for cp in copies:
                cp.wait()
            pltpu.sync_copy(
                rows_v, out_hbm.at[pl.ds(rbase + p * (K * GROUP), K * GROUP)]
            )
            return ()

        lax.fori_loop(0, n_super, super_body, (), unroll=False)

    f = pl.kernel(
        body,
        out_type=jax.ShapeDtypeStruct((n_rows, D), jnp.float32),
        mesh=mesh,
        scratch_types=[
            pltpu.VMEM((G, GROUP), jnp.int32),
            pltpu.VMEM((K * GROUP, D), jnp.float32),
            pltpu.SemaphoreType.DMA,
        ],
        compiler_params=pltpu.CompilerParams(use_tc_tiling_on_sc=False),
    )
    return f(idx2d, table)


def kernel(input, weight):
    Bw, Vw, Dw = weight.shape
    idx = input.reshape(-1).astype(jnp.int32)
    n = idx.shape[0]
    idx2d = idx.reshape(n // GROUP, GROUP)
    out = _emb_lookup(idx2d, weight, Bw)
    return out.reshape(input.shape + (Dw,))


# speedup vs baseline: 15.5350x; 1.0080x over previous
import functools

import jax
import jax.numpy as jnp
from jax import lax
from jax.experimental import pallas as pl
from jax.experimental.pallas import tpu as pltpu
from jax.experimental.pallas import tpu_sc as plsc

NC = 2
NS = 16
NW = NC * NS
GROUP = 128


@functools.partial(jax.jit, static_argnames=("n_batches",))
def _emb_lookup(idx2d, table, n_batches):
    n_groups_total, _ = idx2d.shape
    _, V, D = table.shape
    n_rows = n_groups_total * GROUP
    G = n_groups_total // NW
    per_tile = G * GROUP
    K = 8
    n_super = G // K
    tiles_per_batch = NW // n_batches

    mesh = plsc.VectorSubcoreMesh(core_axis_name="c", subcore_axis_name="s")

    def body(idx_hbm, tab_hbm, out_hbm, idx_v, rows_v, gsem, wsem):
        c = lax.axis_index("c")
        s = lax.axis_index("s")
        wid = s * NC + c
        b = wid // tiles_per_batch
        gbase = wid * G
        rbase = wid * per_tile

        pltpu.sync_copy(idx_hbm.at[pl.ds(gbase, G)], idx_v)

        def do_chunk(p, buf):
            copies = []
            for k in range(K):
                copies.append(
                    pltpu.async_copy(
                        tab_hbm.at[b].at[idx_v.at[p * K + k]],
                        rows_v.at[buf, pl.ds(k * GROUP, GROUP)],
                        gsem,
                    )
                )
            for cp in copies:
                cp.wait()
            pltpu.async_copy(
                rows_v.at[buf],
                out_hbm.at[pl.ds(rbase + p * (K * GROUP), K * GROUP)],
                wsem,
            )

        def drain_write(buf):
            pltpu.make_async_copy(
                rows_v.at[buf], out_hbm.at[pl.ds(rbase, K * GROUP)], wsem
            ).wait()

        do_chunk(0, 0)
        do_chunk(1, 1)

        def super_body(p, _):
            buf = lax.rem(p, 2)
            drain_write(buf)
            do_chunk(p, buf)
            return ()

        lax.fori_loop(2, n_super, super_body, (), unroll=False)
        drain_write(0)
        drain_write(1)

    f = pl.kernel(
        body,
        out_type=jax.ShapeDtypeStruct((n_rows, D), jnp.float32),
        mesh=mesh,
        scratch_types=[
            pltpu.VMEM((G, GROUP), jnp.int32),
            pltpu.VMEM((2, K * GROUP, D), jnp.float32),
            pltpu.SemaphoreType.DMA,
            pltpu.SemaphoreType.DMA,
        ],
        compiler_params=pltpu.CompilerParams(use_tc_tiling_on_sc=False),
    )
    return f(idx2d, table)


def kernel(input, weight):
    Bw, Vw, Dw = weight.shape
    idx = input.reshape(-1).astype(jnp.int32)
    n = idx.shape[0]
    idx2d = idx.reshape(n // GROUP, GROUP)
    out = _emb_lookup(idx2d, weight, Bw)
    return out.reshape(input.shape + (Dw,))


# speedup vs baseline: 29.0749x; 1.8716x over previous
import functools

import jax
import jax.numpy as jnp
from jax import lax
from jax.experimental import pallas as pl
from jax.experimental.pallas import tpu as pltpu
from jax.experimental.pallas import tpu_sc as plsc

NC = 2
NS = 16
NW = NC * NS


@functools.partial(jax.jit, static_argnames=("n_batches",))
def _emb_lookup(idx2d, table, n_batches):
    n_rows, JD = idx2d.shape
    _, V, D = table.shape
    R = n_rows // NW
    CH = 16
    n_chunks = R // CH
    tiles_per_batch = NW // n_batches

    mesh = plsc.VectorSubcoreMesh(core_axis_name="c", subcore_axis_name="s")

    def body(idx_hbm, tab_hbm, out_hbm, idx_v, rows_v, gsem, wsem):
        c = lax.axis_index("c")
        s = lax.axis_index("s")
        wid = s * NC + c
        b = wid // tiles_per_batch
        rbase = wid * R

        pltpu.sync_copy(idx_hbm.at[pl.ds(rbase, R)], idx_v)

        def do_chunk(p, buf):
            copies = []
            for k in range(CH):
                copies.append(
                    pltpu.async_copy(
                        tab_hbm.at[b].at[idx_v.at[p * CH + k]],
                        rows_v.at[buf, k],
                        gsem,
                    )
                )
            for cp in copies:
                cp.wait()
            pltpu.async_copy(
                rows_v.at[buf],
                out_hbm.at[pl.ds(rbase + p * CH, CH)],
                wsem,
            )

        def drain_write(buf):
            pltpu.make_async_copy(
                rows_v.at[buf], out_hbm.at[pl.ds(rbase, CH)], wsem
            ).wait()

        do_chunk(0, 0)
        do_chunk(1, 1)

        def chunk_body(p, _):
            buf = lax.rem(p, 2)
            drain_write(buf)
            do_chunk(p, buf)
            return ()

        lax.fori_loop(2, n_chunks, chunk_body, (), unroll=False)
        drain_write(0)
        drain_write(1)

    f = pl.kernel(
        body,
        out_type=jax.ShapeDtypeStruct((n_rows, JD, D), jnp.float32),
        mesh=mesh,
        scratch_types=[
            pltpu.VMEM((R, JD), jnp.int32),
            pltpu.VMEM((2, CH, JD, D), jnp.float32),
            pltpu.SemaphoreType.DMA,
            pltpu.SemaphoreType.DMA,
        ],
        compiler_params=pltpu.CompilerParams(use_tc_tiling_on_sc=False),
    )
    return f(idx2d, table)


def kernel(input, weight):
    Bw, Vw, Dw = weight.shape
    lead = input.shape[:-1]
    JD = input.shape[-1]
    n_rows = 1
    for d_ in lead:
        n_rows *= d_
    idx2d = input.reshape(n_rows, JD).astype(jnp.int32)
    out = _emb_lookup(idx2d, weight, Bw)
    return out.reshape(input.shape + (Dw,))
